# baseline (speedup 1.0000x reference)
import functools

import numpy as np
import jax
from jax import lax
import jax.numpy as jnp
from jax.experimental import pallas as pl
from jax.experimental.pallas import tpu as pltpu
from jax.experimental.pallas import tpu_sc as plsc

_H = 512
_W = 512
_NPIX = float(_H * _W)
_NUM_BINS = 8
_WIDTH = 255.0 / 8.0

_FR = float(-np.sin(2.0 * np.pi * 5 / 8))
_FC = float(np.cos(2.0 * np.pi * 5 / 8) + 1.0)
_A = _FR * _FC
_B = _FR * _FR
_T = 1.0 - (1.0 - _FR) * _FC
_AT = np.float32(_A / _T)
_BT = np.float32(_B / _T)

_MOB = np.zeros((8, _NUM_BINS), dtype=np.float32)
_MOB[0] = [-1.0, 1.0, 0.0, 0.0, 0.0, 0.0, 0.0, 0.0]
_MOB[1] = [-1.0, 0.0, 1.0, 0.0, 0.0, 0.0, 0.0, 0.0]
_MOB[2] = [-1.0, 0.0, 0.0, 0.0, 1.0, 0.0, 0.0, 0.0]
_MOB[3] = [1.0, -1.0, -1.0, 1.0, 0.0, 0.0, 0.0, 0.0]
_MOB[4] = [1.0, -1.0, 0.0, 0.0, -1.0, 1.0, 0.0, 0.0]
_MOB[5] = [1.0, 0.0, -1.0, 0.0, -1.0, 0.0, 1.0, 0.0]
_MOB[6] = [-1.0, 1.0, 1.0, -1.0, 1.0, -1.0, -1.0, 1.0]
_MOB[7] = [1.0, 0.0, 0.0, 0.0, 0.0, 0.0, 0.0, 0.0]


_PLANES_PER_STEP = 2


def _plane_hist(x, masks, mob):
    last_row, first_col, last_col = masks
    rn = pltpu.roll(x, _H - 1, 0) * last_row
    g = _AT * x + _BT * rn
    gm = pltpu.roll(g, 1, 1) * first_col
    gp = pltpu.roll(g, _W - 1, 1) * last_col
    w = x - _AT * rn

    b5 = (gm >= w).astype(jnp.float32)
    b6 = (rn >= x).astype(jnp.float32)
    b7 = (gp >= w).astype(jnp.float32)
    p56 = b5 * b6
    p57 = b5 * b7
    p67 = b6 * b7
    p567 = p56 * b7

    counts = (
        jnp.sum(b5) * mob[0]
        + jnp.sum(b6) * mob[1]
        + jnp.sum(b7) * mob[2]
        + jnp.sum(p56) * mob[3]
        + jnp.sum(p57) * mob[4]
        + jnp.sum(p67) * mob[5]
        + jnp.sum(p567) * mob[6]
        + _NPIX * mob[7]
    )
    return (counts / np.float32(_WIDTH)) / np.float32(_NPIX)


def _lbp_hist_kernel(x_ref, mob_ref, out_ref):
    rowi = jax.lax.broadcasted_iota(jnp.int32, (_H, 1), 0)
    coli = jax.lax.broadcasted_iota(jnp.int32, (1, _W), 1)
    masks = (
        jnp.where(rowi < _H - 1, 1.0, 0.0).astype(jnp.float32),
        jnp.where(coli > 0, 1.0, 0.0).astype(jnp.float32),
        jnp.where(coli < _W - 1, 1.0, 0.0).astype(jnp.float32),
    )
    mob = mob_ref[...]
    for k in range(_PLANES_PER_STEP):
        out_ref[k, 0] = _plane_hist(x_ref[k], masks, mob)


_LANES = 16
_SC_PLANES = 16
_SC_TASKS = 32
_TASK_ROWS = _H // 2
_SC_CHUNK = 64
_N_CHUNKS = _TASK_ROWS // _SC_CHUNK
_CVEC = _W // _LANES


def _sc_lbp_body(x_hbm, out_hbm, buf_a, buf_b, gbuf, obuf, sem_a, sem_b):
    w_id = lax.axis_index("s") * 2 + lax.axis_index("c")
    plane = w_id // 2
    half = w_id - 2 * plane
    r_base = half * _TASK_ROWS
    zero16 = jnp.zeros((_LANES,), jnp.float32)
    gbuf[pl.ds(0, _LANES)] = zero16
    gbuf[pl.ds(_W + 2 - _LANES, _LANES)] = zero16

    bufs = (buf_a, buf_b)
    sems = (sem_a, sem_b)

    def start(ck):
        r0 = r_base + ck * _SC_CHUNK
        halo = r0 + _SC_CHUNK
        halo = jnp.where(halo < _H, halo, 0)
        cps = [pltpu.async_copy(
            x_hbm.at[plane, pl.ds(r0, _SC_CHUNK), :],
            bufs[ck % 2].at[pl.ds(0, _SC_CHUNK), :],
            sems[ck % 2])]
        cps.append(pltpu.async_copy(
            x_hbm.at[plane, pl.ds(halo, 1), :],
            bufs[ck % 2].at[pl.ds(_SC_CHUNK, 1), :],
            sems[ck % 2]))
        return cps

    def row_body_for(chunk):
        def row_body(rr, acc):
            for c in range(_CVEC):
                xv = chunk[rr, pl.ds(c * _LANES, _LANES)]
                rv = chunk[rr + 1, pl.ds(c * _LANES, _LANES)]
                gbuf[pl.ds(1 + c * _LANES, _LANES)] = _AT * xv + _BT * rv

            s5, s6, s7, s56, s57, s67, s567 = acc
            for c in range(_CVEC):
                xv = chunk[rr, pl.ds(c * _LANES, _LANES)]
                rv = chunk[rr + 1, pl.ds(c * _LANES, _LANES)]
                gm = gbuf[pl.ds(c * _LANES, _LANES)]
                gp = gbuf[pl.ds(c * _LANES + 2, _LANES)]
                w = xv - _AT * rv
                f5 = jnp.where(gm >= w, 1.0, 0.0).astype(jnp.float32)
                f6 = jnp.where(rv >= xv, 1.0, 0.0).astype(jnp.float32)
                f7 = jnp.where(gp >= w, 1.0, 0.0).astype(jnp.float32)
                p56 = f5 * f6
                p67 = f6 * f7
                s5 = s5 + f5
                s6 = s6 + f6
                s7 = s7 + f7
                s56 = s56 + p56
                s57 = s57 + f5 * f7
                s67 = s67 + p67
                s567 = s567 + f5 * p67
            return (s5, s6, s7, s56, s57, s67, s567)
        return row_body

    acc = tuple(jnp.zeros((_LANES,), jnp.float32) for _ in range(7))
    pending = start(0)
    for ck in range(_N_CHUNKS):
        chunk = bufs[ck % 2]
        nxt = start(ck + 1) if ck < _N_CHUNKS - 1 else None
        for cp in pending:
            cp.wait()
        if ck == _N_CHUNKS - 1:
            @pl.when(half == 1)
            def _():
                for c in range(_CVEC):
                    chunk[_SC_CHUNK, pl.ds(c * _LANES, _LANES)] = zero16
        acc = lax.fori_loop(0, _SC_CHUNK, row_body_for(chunk), acc)
        pending = nxt

    s5, s6, s7, s56, s57, s67, s567 = [jnp.sum(a) for a in acc]
    n = jnp.float32(_NPIX)
    cnts = (
        n - s5 - s6 - s7 + s56 + s57 + s67 - s567,
        s5 - s56 - s57 + s567,
        s6 - s56 - s67 + s567,
        s56 - s567,
        s7 - s57 - s67 + s567,
        s57 - s567,
        s67 - s567,
        s567,
    )
    io = lax.iota(jnp.int32, _LANES)
    dv = jnp.zeros((_LANES,), jnp.float32)
    for j, cj in enumerate(cnts):
        dv = jnp.where(io == j, cj, dv)
    obuf[...] = dv
    pltpu.sync_copy(obuf, out_hbm.at[w_id])


_sc_lbp = functools.partial(
    pl.kernel,
    mesh=plsc.VectorSubcoreMesh(core_axis_name="c", subcore_axis_name="s"),
    out_type=jax.ShapeDtypeStruct((_SC_TASKS, _LANES), jnp.float32),
    compiler_params=pltpu.CompilerParams(needs_layout_passes=False),
    scratch_types=[
        pltpu.VMEM((_SC_CHUNK + 1, _W), jnp.float32),
        pltpu.VMEM((_SC_CHUNK + 1, _W), jnp.float32),
        pltpu.VMEM((_W + 2,), jnp.float32),
        pltpu.VMEM((_LANES,), jnp.float32),
        pltpu.SemaphoreType.DMA,
        pltpu.SemaphoreType.DMA,
    ],
)(_sc_lbp_body)


def kernel(x):
    B, C, H, W = x.shape
    planes = x.reshape(B * C, H, W)
    n_sc = _SC_PLANES
    n_tc = B * C - n_sc
    sc_parts = _sc_lbp(planes[:n_sc])
    sc_counts = sc_parts.reshape(n_sc, 2, _LANES).sum(axis=1)
    out_sc = (sc_counts / np.float32(_WIDTH)) / np.float32(_NPIX)
    out_tc = pl.pallas_call(
        _lbp_hist_kernel,
        grid=(n_tc // _PLANES_PER_STEP,),
        in_specs=[
            pl.BlockSpec((_PLANES_PER_STEP, H, W), lambda i: (i, 0, 0)),
            pl.BlockSpec((8, _NUM_BINS), lambda i: (0, 0)),
        ],
        out_specs=pl.BlockSpec(
            (_PLANES_PER_STEP, 1, _NUM_BINS), lambda i: (i, 0, 0)),
        out_shape=jax.ShapeDtypeStruct((n_tc, 1, _NUM_BINS), jnp.float32),
        compiler_params=pltpu.CompilerParams(
            dimension_semantics=("parallel",),
        ),
    )(planes[n_sc:], jnp.asarray(_MOB))
    hist = jnp.concatenate(
        [out_sc[:, :_NUM_BINS], out_tc.reshape(n_tc, _NUM_BINS)], axis=0)
    return hist.reshape(B, C * _NUM_BINS)


# speedup vs baseline: 1.0012x; 1.0012x over previous
import functools

import numpy as np
import jax
from jax import lax
import jax.numpy as jnp
from jax.experimental import pallas as pl
from jax.experimental.pallas import tpu as pltpu
from jax.experimental.pallas import tpu_sc as plsc

_H = 512
_W = 512
_NPIX = float(_H * _W)
_NUM_BINS = 8
_WIDTH = 255.0 / 8.0

_FR = float(-np.sin(2.0 * np.pi * 5 / 8))
_FC = float(np.cos(2.0 * np.pi * 5 / 8) + 1.0)
_A = _FR * _FC
_B = _FR * _FR
_T = 1.0 - (1.0 - _FR) * _FC
_AT = np.float32(_A / _T)
_BT = np.float32(_B / _T)

_MOB = np.zeros((8, _NUM_BINS), dtype=np.float32)
_MOB[0] = [-1.0, 1.0, 0.0, 0.0, 0.0, 0.0, 0.0, 0.0]
_MOB[1] = [-1.0, 0.0, 1.0, 0.0, 0.0, 0.0, 0.0, 0.0]
_MOB[2] = [-1.0, 0.0, 0.0, 0.0, 1.0, 0.0, 0.0, 0.0]
_MOB[3] = [1.0, -1.0, -1.0, 1.0, 0.0, 0.0, 0.0, 0.0]
_MOB[4] = [1.0, -1.0, 0.0, 0.0, -1.0, 1.0, 0.0, 0.0]
_MOB[5] = [1.0, 0.0, -1.0, 0.0, -1.0, 0.0, 1.0, 0.0]
_MOB[6] = [-1.0, 1.0, 1.0, -1.0, 1.0, -1.0, -1.0, 1.0]
_MOB[7] = [1.0, 0.0, 0.0, 0.0, 0.0, 0.0, 0.0, 0.0]


_PLANES_PER_STEP = 2


def _plane_hist(x, masks, mob):
    last_row, first_col, last_col = masks
    rn = pltpu.roll(x, _H - 1, 0) * last_row
    g = _AT * x + _BT * rn
    gm = pltpu.roll(g, 1, 1) * first_col
    gp = pltpu.roll(g, _W - 1, 1) * last_col
    w = x - _AT * rn

    b5 = (gm >= w).astype(jnp.float32)
    b6 = (rn >= x).astype(jnp.float32)
    b7 = (gp >= w).astype(jnp.float32)
    p56 = b5 * b6
    p57 = b5 * b7
    p67 = b6 * b7
    p567 = p56 * b7

    counts = (
        jnp.sum(b5) * mob[0]
        + jnp.sum(b6) * mob[1]
        + jnp.sum(b7) * mob[2]
        + jnp.sum(p56) * mob[3]
        + jnp.sum(p57) * mob[4]
        + jnp.sum(p67) * mob[5]
        + jnp.sum(p567) * mob[6]
        + _NPIX * mob[7]
    )
    return (counts / np.float32(_WIDTH)) / np.float32(_NPIX)


def _lbp_hist_kernel(x_ref, mob_ref, out_ref):
    rowi = jax.lax.broadcasted_iota(jnp.int32, (_H, 1), 0)
    coli = jax.lax.broadcasted_iota(jnp.int32, (1, _W), 1)
    masks = (
        jnp.where(rowi < _H - 1, 1.0, 0.0).astype(jnp.float32),
        jnp.where(coli > 0, 1.0, 0.0).astype(jnp.float32),
        jnp.where(coli < _W - 1, 1.0, 0.0).astype(jnp.float32),
    )
    mob = mob_ref[...]
    for k in range(_PLANES_PER_STEP):
        out_ref[k, 0] = _plane_hist(x_ref[k], masks, mob)


_LANES = 16
_SC_PLANES = 16
_SC_TASKS = 32
_TASK_ROWS = _H // 2
_SC_CHUNK = 64
_N_CHUNKS = _TASK_ROWS // _SC_CHUNK
_CVEC = _W // _LANES


def _sc_lbp_body(x_hbm, out_hbm, buf_a, buf_b, gbuf, obuf, sem_a, sem_b):
    w_id = lax.axis_index("s") * 2 + lax.axis_index("c")
    plane = w_id // 2
    half = w_id - 2 * plane
    r_base = half * _TASK_ROWS
    zero16 = jnp.zeros((_LANES,), jnp.float32)
    gbuf[pl.ds(0, _LANES)] = zero16
    gbuf[pl.ds(_W + 2 - _LANES, _LANES)] = zero16

    bufs = (buf_a, buf_b)
    sems = (sem_a, sem_b)

    def start(ck):
        r0 = r_base + ck * _SC_CHUNK
        halo = r0 + _SC_CHUNK
        halo = jnp.where(halo < _H, halo, 0)
        cps = [pltpu.async_copy(
            x_hbm.at[plane, pl.ds(r0, _SC_CHUNK), :],
            bufs[ck % 2].at[pl.ds(0, _SC_CHUNK), :],
            sems[ck % 2])]
        cps.append(pltpu.async_copy(
            x_hbm.at[plane, pl.ds(halo, 1), :],
            bufs[ck % 2].at[pl.ds(_SC_CHUNK, 1), :],
            sems[ck % 2]))
        return cps

    def row_body_for(chunk):
        def row_body(rr, acc):
            for c in range(_CVEC):
                xv = chunk[rr, pl.ds(c * _LANES, _LANES)]
                rv = chunk[rr + 1, pl.ds(c * _LANES, _LANES)]
                gbuf[pl.ds(1 + c * _LANES, _LANES)] = _AT * xv + _BT * rv

            s5, s6, s7, s56, s57, s67, s567 = acc
            for c in range(_CVEC):
                xv = chunk[rr, pl.ds(c * _LANES, _LANES)]
                rv = chunk[rr + 1, pl.ds(c * _LANES, _LANES)]
                gm = gbuf[pl.ds(c * _LANES, _LANES)]
                gp = gbuf[pl.ds(c * _LANES + 2, _LANES)]
                w = xv - _AT * rv
                f5 = jnp.where(gm >= w, 1.0, 0.0).astype(jnp.float32)
                f6 = jnp.where(rv >= xv, 1.0, 0.0).astype(jnp.float32)
                f7 = jnp.where(gp >= w, 1.0, 0.0).astype(jnp.float32)
                p56 = f5 * f6
                p67 = f6 * f7
                s5 = s5 + f5
                s6 = s6 + f6
                s7 = s7 + f7
                s56 = s56 + p56
                s57 = s57 + f5 * f7
                s67 = s67 + p67
                s567 = s567 + f5 * p67
            return (s5, s6, s7, s56, s57, s67, s567)
        return row_body

    acc = tuple(jnp.zeros((_LANES,), jnp.float32) for _ in range(7))
    pending = start(0)
    for ck in range(_N_CHUNKS):
        chunk = bufs[ck % 2]
        nxt = start(ck + 1) if ck < _N_CHUNKS - 1 else None
        for cp in pending:
            cp.wait()
        if ck == _N_CHUNKS - 1:
            @pl.when(half == 1)
            def _():
                for c in range(_CVEC):
                    chunk[_SC_CHUNK, pl.ds(c * _LANES, _LANES)] = zero16
        acc = lax.fori_loop(0, _SC_CHUNK, row_body_for(chunk), acc)
        pending = nxt

    s5, s6, s7, s56, s57, s67, s567 = [jnp.sum(a) for a in acc]
    n = jnp.float32(_TASK_ROWS * _W)
    cnts = (
        n - s5 - s6 - s7 + s56 + s57 + s67 - s567,
        s5 - s56 - s57 + s567,
        s6 - s56 - s67 + s567,
        s56 - s567,
        s7 - s57 - s67 + s567,
        s57 - s567,
        s67 - s567,
        s567,
    )
    io = lax.iota(jnp.int32, _LANES)
    dv = jnp.zeros((_LANES,), jnp.float32)
    for j, cj in enumerate(cnts):
        dv = jnp.where(io == j, cj, dv)
    obuf[...] = dv
    pltpu.sync_copy(obuf, out_hbm.at[w_id])


_sc_lbp = functools.partial(
    pl.kernel,
    mesh=plsc.VectorSubcoreMesh(core_axis_name="c", subcore_axis_name="s"),
    out_type=jax.ShapeDtypeStruct((_SC_TASKS, _LANES), jnp.float32),
    compiler_params=pltpu.CompilerParams(needs_layout_passes=False),
    scratch_types=[
        pltpu.VMEM((_SC_CHUNK + 1, _W), jnp.float32),
        pltpu.VMEM((_SC_CHUNK + 1, _W), jnp.float32),
        pltpu.VMEM((_W + 2,), jnp.float32),
        pltpu.VMEM((_LANES,), jnp.float32),
        pltpu.SemaphoreType.DMA,
        pltpu.SemaphoreType.DMA,
    ],
)(_sc_lbp_body)


def kernel(x):
    B, C, H, W = x.shape
    planes = x.reshape(B * C, H, W)
    n_sc = _SC_PLANES
    n_tc = B * C - n_sc
    sc_parts = _sc_lbp(planes[:n_sc])
    sc_counts = sc_parts.reshape(n_sc, 2, _LANES).sum(axis=1)
    out_sc = (sc_counts / np.float32(_WIDTH)) / np.float32(_NPIX)
    out_tc = pl.pallas_call(
        _lbp_hist_kernel,
        grid=(n_tc // _PLANES_PER_STEP,),
        in_specs=[
            pl.BlockSpec((_PLANES_PER_STEP, H, W), lambda i: (i, 0, 0)),
            pl.BlockSpec((8, _NUM_BINS), lambda i: (0, 0)),
        ],
        out_specs=pl.BlockSpec(
            (_PLANES_PER_STEP, 1, _NUM_BINS), lambda i: (i, 0, 0)),
        out_shape=jax.ShapeDtypeStruct((n_tc, 1, _NUM_BINS), jnp.float32),
        compiler_params=pltpu.CompilerParams(
            dimension_semantics=("parallel",),
        ),
    )(planes[n_sc:], jnp.asarray(_MOB))
    hist = jnp.concatenate(
        [out_sc[:, :_NUM_BINS], out_tc.reshape(n_tc, _NUM_BINS)], axis=0)
    return hist.reshape(B, C * _NUM_BINS)


# speedup vs baseline: 1.5776x; 1.5756x over previous
import numpy as np
import jax
import jax.numpy as jnp
from jax.experimental import pallas as pl
from jax.experimental.pallas import tpu as pltpu

_H = 512
_W = 512
_NPIX = float(_H * _W)
_NUM_BINS = 8
_WIDTH = 255.0 / 8.0

_FR = float(-np.sin(2.0 * np.pi * 5 / 8))
_FC = float(np.cos(2.0 * np.pi * 5 / 8) + 1.0)
_A = _FR * _FC
_B = _FR * _FR
_T = 1.0 - (1.0 - _FR) * _FC
_AT = np.float32(_A / _T)
_BT = np.float32(_B / _T)

_MOB = np.zeros((8, _NUM_BINS), dtype=np.float32)
_MOB[0] = [-1.0, 1.0, 0.0, 0.0, 0.0, 0.0, 0.0, 0.0]
_MOB[1] = [-1.0, 0.0, 1.0, 0.0, 0.0, 0.0, 0.0, 0.0]
_MOB[2] = [-1.0, 0.0, 0.0, 0.0, 1.0, 0.0, 0.0, 0.0]
_MOB[3] = [1.0, -1.0, -1.0, 1.0, 0.0, 0.0, 0.0, 0.0]
_MOB[4] = [1.0, -1.0, 0.0, 0.0, -1.0, 1.0, 0.0, 0.0]
_MOB[5] = [1.0, 0.0, -1.0, 0.0, -1.0, 0.0, 1.0, 0.0]
_MOB[6] = [-1.0, 1.0, 1.0, -1.0, 1.0, -1.0, -1.0, 1.0]
_MOB[7] = [1.0, 0.0, 0.0, 0.0, 0.0, 0.0, 0.0, 0.0]


_PLANES_PER_STEP = 2


def _plane_hist(x, masks, mob):
    last_row, first_col, last_col = masks
    rn = pltpu.roll(x, _H - 1, 0) * last_row
    g = _AT * x + _BT * rn
    gm = pltpu.roll(g, 1, 1) * first_col
    gp = pltpu.roll(g, _W - 1, 1) * last_col
    w = x - _AT * rn

    b5 = (gm >= w).astype(jnp.float32)
    b6 = (rn >= x).astype(jnp.float32)
    b7 = (gp >= w).astype(jnp.float32)
    p56 = b5 * b6
    p57 = b5 * b7
    p67 = b6 * b7
    p567 = p56 * b7

    counts = (
        jnp.sum(b5) * mob[0]
        + jnp.sum(b6) * mob[1]
        + jnp.sum(b7) * mob[2]
        + jnp.sum(p56) * mob[3]
        + jnp.sum(p57) * mob[4]
        + jnp.sum(p67) * mob[5]
        + jnp.sum(p567) * mob[6]
        + _NPIX * mob[7]
    )
    return (counts / np.float32(_WIDTH)) / np.float32(_NPIX)


def _lbp_hist_kernel(x_ref, mob_ref, out_ref):
    rowi = jax.lax.broadcasted_iota(jnp.int32, (_H, 1), 0)
    coli = jax.lax.broadcasted_iota(jnp.int32, (1, _W), 1)
    masks = (
        jnp.where(rowi < _H - 1, 1.0, 0.0).astype(jnp.float32),
        jnp.where(coli > 0, 1.0, 0.0).astype(jnp.float32),
        jnp.where(coli < _W - 1, 1.0, 0.0).astype(jnp.float32),
    )
    mob = mob_ref[...]
    for k in range(_PLANES_PER_STEP):
        out_ref[k, 0] = _plane_hist(x_ref[k], masks, mob)


def kernel(x):
    B, C, H, W = x.shape
    planes = x.reshape(B * C, H, W)
    n_steps = (B * C) // _PLANES_PER_STEP
    out = pl.pallas_call(
        _lbp_hist_kernel,
        grid=(n_steps,),
        in_specs=[
            pl.BlockSpec((_PLANES_PER_STEP, H, W), lambda i: (i, 0, 0)),
            pl.BlockSpec((8, _NUM_BINS), lambda i: (0, 0)),
        ],
        out_specs=pl.BlockSpec(
            (_PLANES_PER_STEP, 1, _NUM_BINS), lambda i: (i, 0, 0)),
        out_shape=jax.ShapeDtypeStruct((B * C, 1, _NUM_BINS), jnp.float32),
        compiler_params=pltpu.CompilerParams(
            dimension_semantics=("parallel",),
        ),
    )(planes, jnp.asarray(_MOB))
    return out.reshape(B, C * _NUM_BINS)
